# baseline (speedup 1.0000x reference)
import functools

import jax
import jax.numpy as jnp
from jax import lax
from jax.experimental import pallas as pl
from jax.experimental.pallas import tpu as pltpu
from jax.experimental.pallas import tpu_sc as plsc

_NC, _NS = 2, 16
_C = 32


def _sc_body(t, f, x_hbm, len_hbm, marker_hbm, zeros_hbm, out_hbm,
             zbuf, mbuf, lbuf, bbuf):
    nchunks_half = t // _C // _NC
    b = lax.axis_index("s")
    half = lax.axis_index("c")
    c0 = half * nchunks_half

    pltpu.sync_copy(zeros_hbm, zbuf)
    pltpu.sync_copy(marker_hbm, mbuf)
    pltpu.sync_copy(len_hbm, lbuf)

    lvec = lbuf[...]
    lane = lax.broadcasted_iota(jnp.int32, (16,), 0)
    len_b = jnp.sum(jnp.where(lane == b, lvec, 0))

    for i in range(nchunks_half):
        s = (c0 + i) * _C

        @pl.when(s + _C <= len_b)
        def _copy():
            pltpu.sync_copy(x_hbm.at[b, pl.ds(s, _C)],
                            out_hbm.at[b, pl.ds(s, _C)])

        @pl.when(s > len_b)
        def _zero():
            pltpu.sync_copy(zbuf, out_hbm.at[b, pl.ds(s, _C)])

    @pl.when((c0 * _C <= len_b) & (len_b < (c0 + nchunks_half) * _C))
    def _boundary():
        sb = (len_b // _C) * _C
        nv = len_b - sb
        pltpu.sync_copy(x_hbm.at[b, pl.ds(sb, _C)], bbuf)

        def _row_body(r, carry):
            def _grp_body(j, carry2):
                mv = mbuf[pl.ds(j * 16, 16)]
                res = jnp.where(r == nv, mv, jnp.zeros_like(mv))
                bbuf[r, pl.ds(j * 16, 16)] = res
                return carry2
            return lax.fori_loop(0, f // 16, _grp_body, carry)

        lax.fori_loop(nv, _C, _row_body, 0)
        pltpu.sync_copy(bbuf, out_hbm.at[b, pl.ds(sb, _C)])

    @pl.when(half == _NC - 1)
    def _last_row():
        pltpu.sync_copy(zbuf.at[pl.ds(0, 1)], out_hbm.at[b, pl.ds(t, 1)])


def kernel(x, length, marker):
    b, t, f = x.shape
    length = length.astype(jnp.int32)
    zeros_chunk = jnp.zeros((_C, f), dtype=x.dtype)

    mesh = plsc.VectorSubcoreMesh(
        core_axis_name="c", subcore_axis_name="s",
        num_cores=_NC, num_subcores=_NS)

    sc_call = functools.partial(
        pl.kernel,
        out_type=jax.ShapeDtypeStruct((b, t + 1, f), x.dtype),
        mesh=mesh,
        compiler_params=pltpu.CompilerParams(needs_layout_passes=False),
        scratch_types=[
            pltpu.VMEM((_C, f), x.dtype),
            pltpu.VMEM((f,), x.dtype),
            pltpu.VMEM((16,), jnp.int32),
            pltpu.VMEM((_C, f), x.dtype),
        ],
    )(functools.partial(_sc_body, t, f))

    x_eos = sc_call(x, length, marker, zeros_chunk)
    length_eos = length.astype(jnp.float32) + 1.0
    return x_eos, length_eos


# speedup vs baseline: 9.0729x; 9.0729x over previous
import functools

import jax
import jax.numpy as jnp
from jax import lax
from jax.experimental import pallas as pl
from jax.experimental.pallas import tpu as pltpu
from jax.experimental.pallas import tpu_sc as plsc

_NC, _NS = 2, 16
_C = 32
_WAVE = 8


def _sc_body(t, f, x_hbm, len_hbm, marker_hbm, zeros_hbm, out_hbm,
             zbuf, mbuf, lbuf, pbuf, rd_sem, wr_sem, zsem):
    nch = t // _C // _NC
    b = lax.axis_index("s")
    half = lax.axis_index("c")
    c0 = half * nch

    pltpu.sync_copy(zeros_hbm, zbuf)
    pltpu.sync_copy(marker_hbm, mbuf)
    pltpu.sync_copy(len_hbm, lbuf)

    lvec = lbuf[...]
    lane = lax.broadcasted_iota(jnp.int32, (16,), 0)
    len_b = jnp.sum(jnp.where(lane == b, lvec, 0))
    cb = len_b // _C
    ncopy = jnp.clip(cb - c0, 0, nch)
    j0 = jnp.clip(cb + 1 - c0, 0, nch)

    def _chunk(ref, i):
        return ref.at[b, pl.ds((c0 + i) * _C, _C)]

    @pl.when(ncopy > 0)
    def _prime():
        pltpu.async_copy(_chunk(x_hbm, 0), pbuf.at[0], rd_sem.at[0])

    def _copy_body(i, carry):
        p = lax.rem(i, 2)

        @pl.when(i + 1 < ncopy)
        def _prefetch():
            @pl.when(i >= 1)
            def _free_buf():
                pltpu.make_async_copy(
                    pbuf.at[1 - p], _chunk(out_hbm, i - 1),
                    wr_sem.at[1 - p]).wait()
            pltpu.async_copy(_chunk(x_hbm, i + 1), pbuf.at[1 - p],
                             rd_sem.at[1 - p])

        pltpu.make_async_copy(_chunk(x_hbm, i), pbuf.at[p],
                              rd_sem.at[p]).wait()
        pltpu.async_copy(pbuf.at[p], _chunk(out_hbm, i), wr_sem.at[p])
        return carry

    lax.fori_loop(0, ncopy, _copy_body, 0)

    @pl.when(ncopy >= 2)
    def _drain_w2():
        pltpu.make_async_copy(pbuf.at[lax.rem(ncopy, 2)],
                              _chunk(out_hbm, ncopy - 2),
                              wr_sem.at[lax.rem(ncopy, 2)]).wait()

    @pl.when(ncopy >= 1)
    def _drain_w1():
        pltpu.make_async_copy(pbuf.at[lax.rem(ncopy - 1, 2)],
                              _chunk(out_hbm, ncopy - 1),
                              wr_sem.at[lax.rem(ncopy - 1, 2)]).wait()

    for w in range(0, nch, _WAVE):
        for k in range(_WAVE):
            @pl.when(w + k >= j0)
            def _zfire():
                pltpu.async_copy(zbuf, _chunk(out_hbm, w + k), zsem)

        nz_wave = jnp.clip(w + _WAVE - jnp.maximum(j0, w), 0, _WAVE)

        def _zdrain(i, carry):
            pltpu.make_async_copy(zbuf, _chunk(out_hbm, 0), zsem).wait()
            return carry

        lax.fori_loop(0, nz_wave, _zdrain, 0)

    @pl.when((c0 <= cb) & (cb < c0 + nch))
    def _boundary():
        sb = cb * _C
        nv = len_b - sb
        pltpu.sync_copy(x_hbm.at[b, pl.ds(sb, _C)], pbuf.at[0])

        def _row_body(r, carry):
            def _grp_body(j, carry2):
                mv = mbuf[pl.ds(j * 16, 16)]
                res = jnp.where(r == nv, mv, jnp.zeros_like(mv))
                pbuf[0, r, pl.ds(j * 16, 16)] = res
                return carry2
            return lax.fori_loop(0, f // 16, _grp_body, carry)

        lax.fori_loop(nv, _C, _row_body, 0)
        pltpu.sync_copy(pbuf.at[0], out_hbm.at[b, pl.ds(sb, _C)])

    @pl.when(half == _NC - 1)
    def _last_row():
        pltpu.sync_copy(zbuf.at[pl.ds(0, 1)], out_hbm.at[b, pl.ds(t, 1)])


def kernel(x, length, marker):
    b, t, f = x.shape
    length = length.astype(jnp.int32)
    zeros_chunk = jnp.zeros((_C, f), dtype=x.dtype)

    mesh = plsc.VectorSubcoreMesh(
        core_axis_name="c", subcore_axis_name="s",
        num_cores=_NC, num_subcores=_NS)

    sc_call = functools.partial(
        pl.kernel,
        out_type=jax.ShapeDtypeStruct((b, t + 1, f), x.dtype),
        mesh=mesh,
        compiler_params=pltpu.CompilerParams(needs_layout_passes=False),
        scratch_types=[
            pltpu.VMEM((_C, f), x.dtype),
            pltpu.VMEM((f,), x.dtype),
            pltpu.VMEM((16,), jnp.int32),
            pltpu.VMEM((2, _C, f), x.dtype),
            pltpu.SemaphoreType.DMA((2,)),
            pltpu.SemaphoreType.DMA((2,)),
            pltpu.SemaphoreType.DMA,
        ],
    )(functools.partial(_sc_body, t, f))

    x_eos = sc_call(x, length, marker, zeros_chunk)
    length_eos = length.astype(jnp.float32) + 1.0
    return x_eos, length_eos


# speedup vs baseline: 9.1436x; 1.0078x over previous
import functools

import jax
import jax.numpy as jnp
from jax import lax
from jax.experimental import pallas as pl
from jax.experimental.pallas import tpu as pltpu
from jax.experimental.pallas import tpu_sc as plsc

_NC, _NS = 2, 16
_NW = _NC * _NS
_C = 32
_ZMAX = 8


def _sc_body(t, f, nb, x_hbm, len_hbm, marker_hbm, zeros_hbm, out_hbm,
             zbuf, mbuf, lbuf, pbuf, rd_sem, wr_sem, zsem):
    ncb = t // _C
    nslots = nb * ncb // _NW
    b_ax = lax.axis_index("s")
    c_ax = lax.axis_index("c")
    w = b_ax * _NC + c_ax

    pltpu.sync_copy(zeros_hbm, zbuf)
    pltpu.sync_copy(marker_hbm, mbuf)
    pltpu.sync_copy(len_hbm, lbuf)

    lvec = lbuf[...]
    lane = lax.broadcasted_iota(jnp.int32, (16,), 0)
    cbvec = lvec // _C

    def _slot(i):
        g = w + _NW * i
        bi = jnp.minimum(g // ncb, nb - 1)
        ci = g - bi * ncb
        cbi = jnp.sum(jnp.where(lane == bi, cbvec, 0))
        leni = jnp.sum(jnp.where(lane == bi, lvec, 0))
        return bi, ci, cbi, leni

    def _chunk(ref, bi, ci):
        return ref.at[bi, pl.ds(ci * _C, _C)]

    def _wait(sem_ref, dst_bi):
        pltpu.make_async_copy(zbuf, _chunk(out_hbm, dst_bi, 0),
                              sem_ref).wait()

    b0, ch0, cb0, _ = _slot(0)

    @pl.when(ch0 < cb0)
    def _prime():
        pltpu.async_copy(_chunk(x_hbm, b0, ch0), pbuf.at[0], rd_sem.at[0])

    def _body(i, carry):
        w0, w1, zc = carry
        p = lax.rem(i, 2)
        q = 1 - p
        wp = jnp.where(p == 0, w0, w1)
        wq = jnp.where(p == 0, w1, w0)
        bi, ci, cbi, leni = _slot(i)
        is_copy = ci < cbi
        is_zero = ci > cbi

        bn, cn, cbn, _ = _slot(i + 1)
        next_copy = (i + 1 < nslots) & (cn < cbn)

        @pl.when(next_copy)
        def _prefetch():
            @pl.when(wq > 0)
            def _free():
                _wait(wr_sem.at[q], bi)
            pltpu.async_copy(_chunk(x_hbm, bn, cn), pbuf.at[q],
                             rd_sem.at[q])

        wq = jnp.where(next_copy, 0, wq)

        @pl.when(is_copy)
        def _copy():
            pltpu.make_async_copy(_chunk(x_hbm, bi, ci), pbuf.at[p],
                                  rd_sem.at[p]).wait()
            pltpu.async_copy(pbuf.at[p], _chunk(out_hbm, bi, ci),
                             wr_sem.at[p])

        wp = jnp.where(is_copy, 1, wp)

        @pl.when(is_zero & (zc >= _ZMAX))
        def _zthrottle():
            _wait(zsem, bi)

        @pl.when(is_zero)
        def _zfire():
            pltpu.async_copy(zbuf, _chunk(out_hbm, bi, ci), zsem)

        zc = zc - jnp.where(is_zero & (zc >= _ZMAX), 1, 0) \
            + jnp.where(is_zero, 1, 0)

        is_bnd = ci == cbi

        @pl.when(is_bnd & (wp > 0))
        def _bfree():
            _wait(wr_sem.at[p], bi)

        wp = jnp.where(is_bnd, 0, wp)

        @pl.when(is_bnd)
        def _boundary():
            nv = leni - cbi * _C
            pltpu.sync_copy(_chunk(x_hbm, bi, ci), pbuf.at[p])

            def _row_body(r, rcarry):
                def _grp_body(j, gcarry):
                    mv = mbuf[pl.ds(j * 16, 16)]
                    res = jnp.where(r == nv, mv, jnp.zeros_like(mv))
                    pbuf[p, r, pl.ds(j * 16, 16)] = res
                    return gcarry
                return lax.fori_loop(0, f // 16, _grp_body, rcarry)

            lax.fori_loop(nv, _C, _row_body, 0)
            pltpu.sync_copy(pbuf.at[p], _chunk(out_hbm, bi, ci))

        w0 = jnp.where(p == 0, wp, wq)
        w1 = jnp.where(p == 0, wq, wp)
        return w0, w1, zc

    w0, w1, zc = lax.fori_loop(
        0, nslots, _body,
        (jnp.int32(0), jnp.int32(0), jnp.int32(0)))

    @pl.when(w0 > 0)
    def _dw0():
        _wait(wr_sem.at[0], 0)

    @pl.when(w1 > 0)
    def _dw1():
        _wait(wr_sem.at[1], 0)

    def _zdrain(i, carry):
        _wait(zsem, 0)
        return carry

    lax.fori_loop(0, zc, _zdrain, 0)

    @pl.when(w < nb)
    def _last_row():
        pltpu.sync_copy(zbuf.at[pl.ds(0, 1)], out_hbm.at[w, pl.ds(t, 1)])


def kernel(x, length, marker):
    b, t, f = x.shape
    length = length.astype(jnp.int32)
    zeros_chunk = jnp.zeros((_C, f), dtype=x.dtype)

    mesh = plsc.VectorSubcoreMesh(
        core_axis_name="c", subcore_axis_name="s",
        num_cores=_NC, num_subcores=_NS)

    sc_call = functools.partial(
        pl.kernel,
        out_type=jax.ShapeDtypeStruct((b, t + 1, f), x.dtype),
        mesh=mesh,
        compiler_params=pltpu.CompilerParams(needs_layout_passes=False),
        scratch_types=[
            pltpu.VMEM((_C, f), x.dtype),
            pltpu.VMEM((f,), x.dtype),
            pltpu.VMEM((16,), jnp.int32),
            pltpu.VMEM((2, _C, f), x.dtype),
            pltpu.SemaphoreType.DMA((2,)),
            pltpu.SemaphoreType.DMA((2,)),
            pltpu.SemaphoreType.DMA,
        ],
    )(functools.partial(_sc_body, t, f, b))

    x_eos = sc_call(x, length, marker, zeros_chunk)
    length_eos = length.astype(jnp.float32) + 1.0
    return x_eos, length_eos
